# Initial kernel scaffold; baseline (speedup 1.0000x reference)
import jax
import jax.numpy as jnp
from jax.experimental import pallas as pl


def kernel(input_idx, embedding_table):
    raise NotImplementedError("write your pallas kernel here")


# speedup vs baseline: 1.5602x; 1.5602x over previous
import functools

import jax
import jax.numpy as jnp
from jax import lax
from jax.experimental import pallas as pl
from jax.experimental.pallas import tpu as pltpu
from jax.experimental.pallas import tpu_sc as plsc

_NC, _NS = 2, 16
_NW = _NC * _NS
_D = 32
_B = 16384 * 26
_BPW = _B // _NW
_STREAM = 128
_SPC = 8
_CHUNK = _STREAM * _SPC
_NCHUNK = _BPW // _CHUNK

_mesh = plsc.VectorSubcoreMesh(core_axis_name="c", subcore_axis_name="s")


@functools.partial(
    pl.kernel,
    mesh=_mesh,
    out_type=jax.ShapeDtypeStruct((_B, _D), jnp.float32),
    scratch_types=[
        pltpu.VMEM((_BPW,), jnp.int32),
        pltpu.VMEM((_CHUNK, _D), jnp.float32),
        pltpu.SemaphoreType.DMA,
    ],
    compiler_params=pltpu.CompilerParams(use_tc_tiling_on_sc=False),
)
def _embed_gather(idx_hbm, table_hbm, out_hbm, idx_v, rows_v, sem):
    wid = lax.axis_index("s") * _NC + lax.axis_index("c")
    base = wid * _BPW
    pltpu.sync_copy(idx_hbm.at[pl.ds(base, _BPW)], idx_v)

    def chunk_body(c, carry):
        copies = []
        for j in range(_SPC):
            off = c * _CHUNK + j * _STREAM
            copies.append(pltpu.async_copy(
                table_hbm.at[idx_v.at[pl.ds(off, _STREAM)]],
                rows_v.at[pl.ds(j * _STREAM, _STREAM)],
                sem))
        for cp in copies:
            cp.wait()
        pltpu.sync_copy(rows_v, out_hbm.at[pl.ds(base + c * _CHUNK, _CHUNK)])
        return carry

    lax.fori_loop(0, _NCHUNK, chunk_body, 0)


def kernel(input_idx, embedding_table):
    flat = input_idx.reshape(_B)
    out = _embed_gather(flat, embedding_table)
    return out.reshape(16384, 26, _D)


# speedup vs baseline: 1.5806x; 1.0131x over previous
import functools

import jax
import jax.numpy as jnp
from jax import lax
from jax.experimental import pallas as pl
from jax.experimental.pallas import tpu as pltpu
from jax.experimental.pallas import tpu_sc as plsc

_NC, _NS = 2, 16
_NW = _NC * _NS
_D = 32
_NB = 16384
_NCOL = 26
_BPW = _NB // _NW
_CHUNK = 128
_NCHUNK = _BPW // _CHUNK

_mesh = plsc.VectorSubcoreMesh(core_axis_name="c", subcore_axis_name="s")


@functools.partial(
    pl.kernel,
    mesh=_mesh,
    out_type=jax.ShapeDtypeStruct((_NB, _NCOL, _D), jnp.float32),
    scratch_types=[
        pltpu.VMEM((_NCOL, _BPW), jnp.int32),
        pltpu.VMEM((_NCOL, _CHUNK, _D), jnp.float32),
        pltpu.SemaphoreType.DMA,
        pltpu.SemaphoreType.DMA,
    ],
    compiler_params=pltpu.CompilerParams(use_tc_tiling_on_sc=False),
)
def _embed_gather(idxt_hbm, table_hbm, out_hbm, idx_v, rows_v, gsem, osem):
    wid = lax.axis_index("s") * _NC + lax.axis_index("c")
    b0 = wid * _BPW
    pltpu.sync_copy(idxt_hbm.at[:, pl.ds(b0, _BPW)], idx_v)

    def chunk_body(j, carry):
        def fire(c, _):
            pltpu.async_copy(
                table_hbm.at[idx_v.at[c, pl.ds(j * _CHUNK, _CHUNK)]],
                rows_v.at[c],
                gsem)
            return _

        def drain(c, _):
            pltpu.make_async_copy(
                table_hbm.at[idx_v.at[0, pl.ds(0, _CHUNK)]],
                rows_v.at[0],
                gsem).wait()
            return _

        def put(c, _):
            pltpu.async_copy(
                rows_v.at[c],
                out_hbm.at[pl.ds(b0 + j * _CHUNK, _CHUNK), c, :],
                osem)
            return _

        def put_drain(c, _):
            pltpu.make_async_copy(
                rows_v.at[0],
                out_hbm.at[pl.ds(b0, _CHUNK), 0, :],
                osem).wait()
            return _

        lax.fori_loop(0, _NCOL, fire, 0)
        lax.fori_loop(0, _NCOL, drain, 0)
        lax.fori_loop(0, _NCOL, put, 0)
        lax.fori_loop(0, _NCOL, put_drain, 0)
        return carry

    lax.fori_loop(0, _NCHUNK, chunk_body, 0)


def kernel(input_idx, embedding_table):
    return _embed_gather(input_idx.T, embedding_table)
